# baseline (speedup 1.0000x reference)
import functools

import jax
import jax.numpy as jnp
from jax import lax
from jax.experimental import pallas as pl
from jax.experimental.pallas import tpu as pltpu
from jax.experimental.pallas import tpu_sc as plsc

DIM = 64
RANK = 2

NC = 2
NS = 16
NW = NC * NS
LANES = 16
CHUNK = 128
STRIPE = 8
NSTAGE = 8


C_SLAB = 128
C_GRP = C_SLAB * RANK // LANES
QUAD = 4
AL_ROWS = 15632


def _sc_compact_body(v, a_hbm, al_hbm, st, cb, sem_s, sem_o):
    wid = lax.axis_index("s") * NC + lax.axis_index("c")
    out_pw = (v // DIM) // NW
    rows_pw = out_pw * DIM
    nslab = rows_pw // C_SLAB
    nquad = nslab // QUAD
    a_base = wid * rows_pw
    o_base = wid * out_pw
    iota = lax.iota(jnp.int32, LANES)

    def in_copy(t, b):
        return pltpu.make_async_copy(
            a_hbm.at[pl.ds(a_base + t * C_SLAB, C_SLAB)], st.at[b], sem_s)

    def out_copy(q, b):
        return pltpu.make_async_copy(
            cb.at[b],
            al_hbm.at[pl.ds(o_base + q * (QUAD * C_SLAB // DIM),
                            QUAD * C_SLAB // DIM)],
            sem_o)

    def extract_slab(sb, cbuf, k):
        for g in range(C_GRP):
            row = (g << 3) + (iota >> 1)
            col = iota & 1
            gg = k * C_GRP + g
            cb[cbuf, gg >> 3, pl.ds((gg & 7) * LANES, LANES)] = (
                plsc.load_gather(st.at[sb], [row, col]))

    NSLOT = 6

    def do_quad(q, cbuf, slot0, i):
        for k in range(QUAD):
            t = QUAD * q + k
            slot = (slot0 + k) % NSLOT
            in_copy(t, slot).wait()
            if k == 0:
                @pl.when(i > 0)
                def _w():
                    out_copy(q - 3, cbuf).wait()
            extract_slab(slot, cbuf, k)
            nxt = t + NSLOT

            @pl.when(nxt < nslab)
            def _f():
                in_copy(nxt, slot).start()
        out_copy(q, cbuf).start()

    for k in range(NSLOT):
        in_copy(k, k).start()

    def step(i, c):
        do_quad(3 * i, 0, 0, i)
        do_quad(3 * i + 1, 1, 4, i)
        do_quad(3 * i + 2, 2, 2, i)
        return c

    lax.fori_loop(0, nquad // 3, step, 0)
    do_quad(nquad - 1, 0, 0, 1)
    out_copy(nquad - 3, 1).wait()
    out_copy(nquad - 2, 2).wait()
    out_copy(nquad - 1, 0).wait()

    split = NW * rows_pw

    @pl.when(wid == 0)
    def _tail0():
        for k in range(QUAD):
            pltpu.sync_copy(a_hbm.at[pl.ds(split + k * C_SLAB, C_SLAB)],
                            st.at[0])
            extract_slab(0, 0, k)
        pltpu.sync_copy(cb.at[0], al_hbm.at[pl.ds(NW * out_pw, 8)])

    @pl.when(wid == 1)
    def _tail1():
        pltpu.sync_copy(a_hbm.at[pl.ds(split + QUAD * C_SLAB, DIM)],
                        st.at[0].at[pl.ds(0, DIM)])
        for g in range(C_GRP // 2):
            row = (g << 3) + (iota >> 1)
            col = iota & 1
            cb[0, 0, pl.ds(g * LANES, LANES)] = plsc.load_gather(
                st.at[0], [row, col])
        pltpu.sync_copy(cb.at[0], al_hbm.at[pl.ds(NW * out_pw + 8, 8)])


def _sc_compact(a):
    v = a.shape[0]
    mesh = plsc.VectorSubcoreMesh(core_axis_name="c", subcore_axis_name="s")
    return pl.kernel(
        functools.partial(_sc_compact_body, v),
        out_type=jax.ShapeDtypeStruct((AL_ROWS, 2 * DIM), jnp.float32),
        mesh=mesh,
        scratch_types=[
            pltpu.VMEM((6, C_SLAB, RANK), jnp.float32),
            pltpu.VMEM((3, QUAD * C_SLAB // DIM, 2 * DIM), jnp.float32),
            pltpu.SemaphoreType.DMA,
            pltpu.SemaphoreType.DMA,
        ],
        compiler_params=pltpu.CompilerParams(needs_layout_passes=False),
    )(a)


def _sc_gather_body(nchunk, idx_hbm, a8, g_hbm, idx_v, sidx_v,
                    rv, cb, sem_g, sem_o):
    wid = lax.axis_index("s") * NC + lax.axis_index("c")
    spc = nchunk // NSTAGE
    rows_ps = spc * CHUNK
    gps = rows_ps // LANES
    ngrp = nchunk * CHUNK // LANES

    pltpu.sync_copy(idx_hbm.at[pl.ds(wid * nchunk, nchunk)], idx_v)

    def pre(g, c):
        di = g >> 3
        off = (g & 7) << 4
        x = idx_v[di, pl.ds(off, LANES)]
        sidx_v[di, pl.ds(off, LANES)] = x >> 2
        return c

    lax.fori_loop(0, ngrp, pre, 0)

    def fire_stage(s, b):
        def fire(j, c):
            pltpu.make_async_copy(
                a8.at[sidx_v.at[s * spc + j]],
                rv.at[b].at[pl.ds(j * CHUNK, CHUNK)],
                sem_g,
            ).start()
            return c
        lax.fori_loop(0, spc, fire, 0)

    def drain_stage(s, b):
        def drain(j, c):
            pltpu.make_async_copy(
                a8.at[sidx_v.at[s * spc + j]],
                rv.at[b].at[pl.ds(j * CHUNK, CHUNK)],
                sem_g,
            ).wait()
            return c
        lax.fori_loop(0, spc, drain, 0)

    def out_copy(s, b):
        base = (wid * nchunk * CHUNK + s * rows_ps) * RANK // (2 * DIM)
        return pltpu.make_async_copy(
            cb.at[b],
            g_hbm.at[pl.ds(base, rows_ps * RANK // (2 * DIM))],
            sem_o)

    iota = lax.iota(jnp.int32, LANES)

    def extract_stage(s, b):
        def ext(g, c):
            gg = s * gps + g
            di = gg >> 3
            off = (gg & 7) << 4
            x = idx_v[di, pl.ds(off, LANES)]
            col0 = (x & 3) << 1
            row = (g << 4) + iota
            r = g >> 2
            c0 = (g & 3) << 5
            cb[b, r, pl.ds(c0, LANES)] = plsc.load_gather(
                rv.at[b], [row, col0])
            cb[b, r, pl.ds(c0 + LANES, LANES)] = plsc.load_gather(
                rv.at[b], [row, col0 + 1])
            return c
        lax.fori_loop(0, gps, ext, 0)

    fire_stage(0, 0)
    for s in range(NSTAGE):
        b = s % 2
        drain_stage(s, b)
        if s + 1 < NSTAGE:
            fire_stage(s + 1, 1 - b)
        if s >= 2:
            out_copy(s - 2, b).wait()
        extract_stage(s, b)
        out_copy(s, b).start()
    out_copy(NSTAGE - 2, NSTAGE % 2).wait()
    out_copy(NSTAGE - 1, 1 - NSTAGE % 2).wait()


def _sc_gather(idx2d, a8):
    n_rows = idx2d.shape[0]
    nchunk = n_rows // NW
    n = n_rows * CHUNK
    spc = nchunk // NSTAGE
    rows_ps = spc * CHUNK
    gps = rows_ps // LANES
    mesh = plsc.VectorSubcoreMesh(core_axis_name="c", subcore_axis_name="s")
    return pl.kernel(
        functools.partial(_sc_gather_body, nchunk),
        out_type=jax.ShapeDtypeStruct((n * RANK // (2 * DIM), 2 * DIM),
                                      jnp.float32),
        mesh=mesh,
        scratch_types=[
            pltpu.VMEM((nchunk, CHUNK), jnp.int32),
            pltpu.VMEM((nchunk, CHUNK), jnp.int32),
            pltpu.VMEM((2, rows_ps, STRIPE), jnp.float32),
            pltpu.VMEM((2, gps // 4, 2 * DIM), jnp.float32),
            pltpu.SemaphoreType.DMA,
            pltpu.SemaphoreType.DMA,
        ],
        compiler_params=pltpu.CompilerParams(use_tc_tiling_on_sc=False,
                                             needs_layout_passes=False),
    )(idx2d, a8)


def _tc_expand_body(blkr, g_ref, w_ref, o_ref):
    m = jnp.dot(g_ref[...], w_ref[...], preferred_element_type=jnp.float32)
    o_ref[...] = m.reshape(blkr, DIM, DIM)


def _tc_expand(gr, w, blkr=128):
    r = gr.shape[0]
    wd = w.shape[1]
    grid = r // blkr
    return pl.pallas_call(
        functools.partial(_tc_expand_body, blkr),
        grid=(grid,),
        in_specs=[
            pl.BlockSpec((blkr, 128), lambda i: (i, 0)),
            pl.BlockSpec((128, wd), lambda i: (0, 0)),
        ],
        out_specs=pl.BlockSpec((blkr, DIM, DIM), lambda i: (i, 0, 0)),
        out_shape=jax.ShapeDtypeStruct((r, DIM, DIM), jnp.float32),
    )(gr, w)


def _make_w2(b):
    w = jnp.kron(jnp.eye(DIM, dtype=jnp.float32), b)
    c = jnp.arange(2 * DIM)
    perm = 2 * (16 * (c >> 5) + (c & 15)) + ((c >> 4) & 1)
    return w[perm, :]


def kernel(ids, A, B):
    bsz, seq = ids.shape
    n = bsz * seq
    idx2d = ids.reshape(n // CHUNK, CHUNK)
    al = _sc_compact(A)
    a8 = al.reshape(AL_ROWS * LANES, STRIPE)
    g = _sc_gather(idx2d, a8)
    out = _tc_expand(g, _make_w2(B))
    return out.reshape(bsz, seq, DIM)


# speedup vs baseline: 1.0117x; 1.0117x over previous
import functools

import jax
import jax.numpy as jnp
from jax import lax
from jax.experimental import pallas as pl
from jax.experimental.pallas import tpu as pltpu
from jax.experimental.pallas import tpu_sc as plsc

DIM = 64
RANK = 2

NC = 2
NS = 16
NW = NC * NS
LANES = 16
CHUNK = 128
STRIPE = 8
NSTAGE = 8


C_SLAB = 128
C_GRP = C_SLAB * RANK // LANES
QUAD = 4
AL_ROWS = 15632


def _sc_compact_body(v, a_hbm, al_hbm, st, cb, sem_s, sem_o):
    wid = lax.axis_index("s") * NC + lax.axis_index("c")
    out_pw = (v // DIM) // NW
    rows_pw = out_pw * DIM
    nslab = rows_pw // C_SLAB
    nquad = nslab // QUAD
    a_base = wid * rows_pw
    o_base = wid * out_pw
    iota = lax.iota(jnp.int32, LANES)

    def in_copy(t, b):
        return pltpu.make_async_copy(
            a_hbm.at[pl.ds(a_base + t * C_SLAB, C_SLAB)], st.at[b], sem_s)

    def out_copy(q, b):
        return pltpu.make_async_copy(
            cb.at[b],
            al_hbm.at[pl.ds(o_base + q * (QUAD * C_SLAB // DIM),
                            QUAD * C_SLAB // DIM)],
            sem_o)

    def extract_slab(sb, cbuf, k):
        for g in range(C_GRP):
            row = (g << 3) + (iota >> 1)
            col = iota & 1
            gg = k * C_GRP + g
            cb[cbuf, gg >> 3, pl.ds((gg & 7) * LANES, LANES)] = (
                plsc.load_gather(st.at[sb], [row, col]))

    def do_quad(q, cbuf, i):
        for k in range(QUAD):
            t = QUAD * q + k
            in_copy(t, k).wait()
            if k == 0:
                @pl.when(i > 0)
                def _w():
                    out_copy(q - 2, cbuf).wait()
            extract_slab(k, cbuf, k)
            nxt = t + QUAD

            @pl.when(nxt < nslab)
            def _f():
                in_copy(nxt, k).start()
        out_copy(q, cbuf).start()

    for k in range(QUAD):
        in_copy(k, k).start()

    def step(i, c):
        do_quad(2 * i, 0, i)
        do_quad(2 * i + 1, 1, i)
        return c

    lax.fori_loop(0, nquad // 2, step, 0)
    do_quad(nquad - 1, 0, 1)
    out_copy(nquad - 2, 1).wait()
    out_copy(nquad - 1, 0).wait()

    split = NW * rows_pw

    @pl.when(wid == 0)
    def _tail0():
        for k in range(QUAD):
            pltpu.sync_copy(a_hbm.at[pl.ds(split + k * C_SLAB, C_SLAB)],
                            st.at[0])
            extract_slab(0, 0, k)
        pltpu.sync_copy(cb.at[0], al_hbm.at[pl.ds(NW * out_pw, 8)])

    @pl.when(wid == 1)
    def _tail1():
        pltpu.sync_copy(a_hbm.at[pl.ds(split + QUAD * C_SLAB, DIM)],
                        st.at[0].at[pl.ds(0, DIM)])
        for g in range(C_GRP // 2):
            row = (g << 3) + (iota >> 1)
            col = iota & 1
            cb[0, 0, pl.ds(g * LANES, LANES)] = plsc.load_gather(
                st.at[0], [row, col])
        pltpu.sync_copy(cb.at[0], al_hbm.at[pl.ds(NW * out_pw + 8, 8)])


def _sc_compact(a):
    v = a.shape[0]
    mesh = plsc.VectorSubcoreMesh(core_axis_name="c", subcore_axis_name="s")
    return pl.kernel(
        functools.partial(_sc_compact_body, v),
        out_type=jax.ShapeDtypeStruct((AL_ROWS, 2 * DIM), jnp.float32),
        mesh=mesh,
        scratch_types=[
            pltpu.VMEM((QUAD, C_SLAB, RANK), jnp.float32),
            pltpu.VMEM((2, QUAD * C_SLAB // DIM, 2 * DIM), jnp.float32),
            pltpu.SemaphoreType.DMA,
            pltpu.SemaphoreType.DMA,
        ],
        compiler_params=pltpu.CompilerParams(needs_layout_passes=False),
    )(a)


def _sc_gather_body(nchunk, idx_hbm, a8, g_hbm, idx_v, sidx_v,
                    rv, cb, sem_g, sem_o):
    wid = lax.axis_index("s") * NC + lax.axis_index("c")
    spc = nchunk // NSTAGE
    rows_ps = spc * CHUNK
    gps = rows_ps // LANES
    ngrp = nchunk * CHUNK // LANES

    pltpu.sync_copy(idx_hbm.at[pl.ds(wid * nchunk, nchunk)], idx_v)

    def pre(g, c):
        di = g >> 3
        off = (g & 7) << 4
        x = idx_v[di, pl.ds(off, LANES)]
        sidx_v[di, pl.ds(off, LANES)] = x >> 2
        return c

    lax.fori_loop(0, ngrp, pre, 0)

    def fire_stage(s, b):
        def fire(j, c):
            pltpu.make_async_copy(
                a8.at[sidx_v.at[s * spc + j]],
                rv.at[b].at[pl.ds(j * CHUNK, CHUNK)],
                sem_g,
            ).start()
            return c
        lax.fori_loop(0, spc, fire, 0)

    def drain_stage(s, b):
        def drain(j, c):
            pltpu.make_async_copy(
                a8.at[sidx_v.at[s * spc + j]],
                rv.at[b].at[pl.ds(j * CHUNK, CHUNK)],
                sem_g,
            ).wait()
            return c
        lax.fori_loop(0, spc, drain, 0)

    def out_copy(s, b):
        base = (wid * nchunk * CHUNK + s * rows_ps) * RANK // (2 * DIM)
        return pltpu.make_async_copy(
            cb.at[b],
            g_hbm.at[pl.ds(base, rows_ps * RANK // (2 * DIM))],
            sem_o)

    iota = lax.iota(jnp.int32, LANES)

    def extract_stage(s, b):
        def ext(g, c):
            gg = s * gps + g
            di = gg >> 3
            off = (gg & 7) << 4
            x = idx_v[di, pl.ds(off, LANES)]
            col0 = (x & 3) << 1
            row = (g << 4) + iota
            r = g >> 2
            c0 = (g & 3) << 5
            cb[b, r, pl.ds(c0, LANES)] = plsc.load_gather(
                rv.at[b], [row, col0])
            cb[b, r, pl.ds(c0 + LANES, LANES)] = plsc.load_gather(
                rv.at[b], [row, col0 + 1])
            return c
        lax.fori_loop(0, gps, ext, 0)

    fire_stage(0, 0)
    for s in range(NSTAGE):
        b = s % 2
        drain_stage(s, b)
        if s + 1 < NSTAGE:
            fire_stage(s + 1, 1 - b)
        if s >= 2:
            out_copy(s - 2, b).wait()
        extract_stage(s, b)
        out_copy(s, b).start()
    out_copy(NSTAGE - 2, NSTAGE % 2).wait()
    out_copy(NSTAGE - 1, 1 - NSTAGE % 2).wait()


def _sc_gather(idx2d, a8):
    n_rows = idx2d.shape[0]
    nchunk = n_rows // NW
    n = n_rows * CHUNK
    spc = nchunk // NSTAGE
    rows_ps = spc * CHUNK
    gps = rows_ps // LANES
    mesh = plsc.VectorSubcoreMesh(core_axis_name="c", subcore_axis_name="s")
    return pl.kernel(
        functools.partial(_sc_gather_body, nchunk),
        out_type=jax.ShapeDtypeStruct((n * RANK // (2 * DIM), 2 * DIM),
                                      jnp.float32),
        mesh=mesh,
        scratch_types=[
            pltpu.VMEM((nchunk, CHUNK), jnp.int32),
            pltpu.VMEM((nchunk, CHUNK), jnp.int32),
            pltpu.VMEM((2, rows_ps, STRIPE), jnp.float32),
            pltpu.VMEM((2, gps // 4, 2 * DIM), jnp.float32),
            pltpu.SemaphoreType.DMA,
            pltpu.SemaphoreType.DMA,
        ],
        compiler_params=pltpu.CompilerParams(use_tc_tiling_on_sc=False,
                                             needs_layout_passes=False),
    )(idx2d, a8)


def _tc_expand_body(blkr, g_ref, w_ref, o_ref):
    m = jnp.dot(g_ref[...], w_ref[...], preferred_element_type=jnp.float32)
    o_ref[...] = m.reshape(blkr, DIM, DIM)


def _tc_expand(gr, w, blkr=128):
    r = gr.shape[0]
    wd = w.shape[1]
    grid = r // blkr
    return pl.pallas_call(
        functools.partial(_tc_expand_body, blkr),
        grid=(grid,),
        in_specs=[
            pl.BlockSpec((blkr, 128), lambda i: (i, 0)),
            pl.BlockSpec((128, wd), lambda i: (0, 0)),
        ],
        out_specs=pl.BlockSpec((blkr, DIM, DIM), lambda i: (i, 0, 0)),
        out_shape=jax.ShapeDtypeStruct((r, DIM, DIM), jnp.float32),
    )(gr, w)


def _make_w2(b):
    w = jnp.kron(jnp.eye(DIM, dtype=jnp.float32), b)
    c = jnp.arange(2 * DIM)
    perm = 2 * (16 * (c >> 5) + (c & 15)) + ((c >> 4) & 1)
    return w[perm, :]


def kernel(ids, A, B):
    bsz, seq = ids.shape
    n = bsz * seq
    idx2d = ids.reshape(n // CHUNK, CHUNK)
    al = _sc_compact(A)
    a8 = al.reshape(AL_ROWS * LANES, STRIPE)
    g = _sc_gather(idx2d, a8)
    out = _tc_expand(g, _make_w2(B))
    return out.reshape(bsz, seq, DIM)
